# Initial kernel scaffold; baseline (speedup 1.0000x reference)
import functools

import jax
import jax.numpy as jnp
from jax import lax
from jax.experimental import pallas as pl
from jax.experimental.pallas import tpu as pltpu
from jax.experimental.pallas import tpu_sc as plsc

N = 10000
E = 320000
D = 128
DE = 16
H = 128

NC = 2
NS = 16
NW = NC * NS
EPW = E // NW
CH = 80
NIT = EPW // CH
RPS = N // NS

_LOG2 = 0.6931471805599453


def _ssp(x):
    return jnp.maximum(x, 0.0) + jnp.log1p(jnp.exp(-jnp.abs(x))) - _LOG2


def _node_proj_body(nf_ref, w_ref, b_ref, out_ref):
    out_ref[...] = (
        jnp.dot(nf_ref[...], w_ref[...], preferred_element_type=jnp.float32)
        + b_ref[...]
    )


def _edge_mlp_body(ef_ref, w1_ref, b1_ref, w2_ref, b2_ref, out_ref):
    t = jnp.dot(ef_ref[...], w1_ref[...], preferred_element_type=jnp.float32)
    t = _ssp(t + b1_ref[...])
    t = jnp.dot(t, w2_ref[...], preferred_element_type=jnp.float32)
    out_ref[...] = _ssp(t + b2_ref[...])


def _out_proj_body(p0_ref, p1_ref, wpo_ref, bpo_ref, wout_ref, bout_ref, out_ref):
    agg = p0_ref[...] + p1_ref[...]
    h = _ssp(
        jnp.dot(agg, wpo_ref[...], preferred_element_type=jnp.float32)
        + bpo_ref[...]
    )
    out_ref[...] = (
        jnp.dot(h, wout_ref[...], preferred_element_type=jnp.float32)
        + bout_ref[...]
    )


RN = 2000
RE = 2000


def _node_proj(nf, w, b):
    return pl.pallas_call(
        _node_proj_body,
        grid=(N // RN,),
        in_specs=[
            pl.BlockSpec((RN, D), lambda i: (i, 0)),
            pl.BlockSpec((D, H), lambda i: (0, 0)),
            pl.BlockSpec((1, H), lambda i: (0, 0)),
        ],
        out_specs=pl.BlockSpec((RN, H), lambda i: (i, 0)),
        out_shape=jax.ShapeDtypeStruct((N, H), jnp.float32),
    )(nf, w, b)


def _edge_mlp(ef, w1, b1, w2, b2):
    return pl.pallas_call(
        _edge_mlp_body,
        grid=(E // RE,),
        in_specs=[
            pl.BlockSpec((RE, DE), lambda i: (i, 0)),
            pl.BlockSpec((DE, H), lambda i: (0, 0)),
            pl.BlockSpec((1, H), lambda i: (0, 0)),
            pl.BlockSpec((H, H), lambda i: (0, 0)),
            pl.BlockSpec((1, H), lambda i: (0, 0)),
        ],
        out_specs=pl.BlockSpec((RE, H), lambda i: (i, 0)),
        out_shape=jax.ShapeDtypeStruct((E, H), jnp.float32),
    )(ef, w1, b1, w2, b2)


def _out_proj(p0, p1, wpo, bpo, wout, bout):
    return pl.pallas_call(
        _out_proj_body,
        grid=(N // RN,),
        in_specs=[
            pl.BlockSpec((RN, H), lambda i: (i, 0)),
            pl.BlockSpec((RN, H), lambda i: (i, 0)),
            pl.BlockSpec((H, D), lambda i: (0, 0)),
            pl.BlockSpec((1, D), lambda i: (0, 0)),
            pl.BlockSpec((D, D), lambda i: (0, 0)),
            pl.BlockSpec((1, D), lambda i: (0, 0)),
        ],
        out_specs=pl.BlockSpec((RN, D), lambda i: (i, 0)),
        out_shape=jax.ShapeDtypeStruct((N, D), jnp.float32),
    )(p0, p1, wpo, bpo, wout, bout)


def _sc_gather_mul_scatter(hv, he, src, dst, zinit):
    mesh = plsc.VectorSubcoreMesh(core_axis_name="c", subcore_axis_name="s")

    @functools.partial(
        pl.kernel,
        mesh=mesh,
        out_type=[
            jax.ShapeDtypeStruct((N, H), jnp.float32),
            jax.ShapeDtypeStruct((N, H), jnp.float32),
        ],
        scratch_types=[
            pltpu.VMEM((CH,), jnp.int32),
            pltpu.VMEM((CH,), jnp.int32),
            pltpu.VMEM((CH, H), jnp.float32),
            pltpu.VMEM((CH, H), jnp.float32),
            pltpu.VMEM_SHARED((N, H), jnp.float32),
            pltpu.SemaphoreType.DMA,
        ],
    )
    def k(hv_hbm, he_hbm, src_hbm, dst_hbm, z_hbm, out0, out1,
          src_v, dst_v, he_v, hvr_v, agg_sh, sem):
        c = lax.axis_index("c")
        s = lax.axis_index("s")
        wid = s * NC + c
        row0 = s * RPS
        pltpu.sync_copy(z_hbm.at[pl.ds(row0, RPS)], agg_sh.at[pl.ds(row0, RPS)])
        plsc.subcore_barrier()

        def chunk(it, carry):
            base = wid * EPW + it * CH
            pltpu.sync_copy(src_hbm.at[pl.ds(base, CH)], src_v)
            pltpu.sync_copy(dst_hbm.at[pl.ds(base, CH)], dst_v)
            pltpu.async_copy(hv_hbm.at[src_v], hvr_v, sem).wait()
            pltpu.sync_copy(he_hbm.at[pl.ds(base, CH)], he_v)

            def row(r, cr):
                for j in range(H // 16):
                    sl = pl.ds(j * 16, 16)
                    hvr_v[r, sl] = hvr_v[r, sl] * he_v[r, sl]
                return cr

            lax.fori_loop(0, CH, row, 0)
            pltpu.sync_copy(hvr_v, agg_sh.at[dst_v], add=True)
            return carry

        lax.fori_loop(0, NIT, chunk, 0)
        plsc.subcore_barrier()

        @pl.when(c == 0)
        def _():
            pltpu.sync_copy(agg_sh.at[pl.ds(row0, RPS)], out0.at[pl.ds(row0, RPS)])

        @pl.when(c == 1)
        def _():
            pltpu.sync_copy(agg_sh.at[pl.ds(row0, RPS)], out1.at[pl.ds(row0, RPS)])

    return k(hv, he, src, dst, zinit)


def kernel(node_feats, edge_feats, edge_index, Wpe1, bpe1, Wpe2, bpe2,
           Wpn, bpn, Wpo, bpo, Wout, bout):
    src = edge_index[0]
    dst = edge_index[1]
    hv = _node_proj(node_feats, Wpn, bpn.reshape(1, H))
    he = _edge_mlp(edge_feats, Wpe1, bpe1.reshape(1, H), Wpe2, bpe2.reshape(1, H))
    zinit = jnp.zeros((N, H), jnp.float32)
    p0, p1 = _sc_gather_mul_scatter(hv, he, src, dst, zinit)
    return _out_proj(p0, p1, Wpo, bpo.reshape(1, H), Wout, bout.reshape(1, D))


# speedup vs baseline: 2.2100x; 2.2100x over previous
import functools

import jax
import jax.numpy as jnp
from jax import lax
from jax.experimental import pallas as pl
from jax.experimental.pallas import tpu as pltpu
from jax.experimental.pallas import tpu_sc as plsc

N = 10000
E = 320000
D = 128
DE = 16
H = 128

NC = 2
NS = 16
NW = NC * NS
EPW = E // NW
CH = 80
NIT = EPW // CH
ZR = 624
ZR_LAST = N - (NS - 1) * ZR

_LOG2 = 0.6931471805599453


def _ssp(x):
    return jnp.maximum(x, 0.0) + jnp.log1p(jnp.exp(-jnp.abs(x))) - _LOG2


def _node_proj_body(nf_ref, w_ref, b_ref, out_ref):
    out_ref[...] = (
        jnp.dot(nf_ref[...], w_ref[...], preferred_element_type=jnp.float32)
        + b_ref[...]
    )


def _edge_mlp_body(ef_ref, w1_ref, b1_ref, w2_ref, b2_ref, out_ref):
    t = jnp.dot(ef_ref[...], w1_ref[...], preferred_element_type=jnp.float32)
    t = _ssp(t + b1_ref[...])
    t = jnp.dot(t, w2_ref[...], preferred_element_type=jnp.float32)
    out_ref[...] = _ssp(t + b2_ref[...])


def _out_proj_body(p0_ref, p1_ref, wpo_ref, bpo_ref, wout_ref, bout_ref, out_ref):
    agg = p0_ref[...] + p1_ref[...]
    h = _ssp(
        jnp.dot(agg, wpo_ref[...], preferred_element_type=jnp.float32)
        + bpo_ref[...]
    )
    out_ref[...] = (
        jnp.dot(h, wout_ref[...], preferred_element_type=jnp.float32)
        + bout_ref[...]
    )


RN = 2000
RE = 2000


def _node_proj(nf, w, b):
    return pl.pallas_call(
        _node_proj_body,
        grid=(N // RN,),
        in_specs=[
            pl.BlockSpec((RN, D), lambda i: (i, 0)),
            pl.BlockSpec((D, H), lambda i: (0, 0)),
            pl.BlockSpec((1, H), lambda i: (0, 0)),
        ],
        out_specs=pl.BlockSpec((RN, H), lambda i: (i, 0)),
        out_shape=jax.ShapeDtypeStruct((N, H), jnp.float32),
    )(nf, w, b)


def _edge_mlp(ef, w1, b1, w2, b2):
    return pl.pallas_call(
        _edge_mlp_body,
        grid=(E // RE,),
        in_specs=[
            pl.BlockSpec((RE, DE), lambda i: (i, 0)),
            pl.BlockSpec((DE, H), lambda i: (0, 0)),
            pl.BlockSpec((1, H), lambda i: (0, 0)),
            pl.BlockSpec((H, H), lambda i: (0, 0)),
            pl.BlockSpec((1, H), lambda i: (0, 0)),
        ],
        out_specs=pl.BlockSpec((RE, H), lambda i: (i, 0)),
        out_shape=jax.ShapeDtypeStruct((E, H), jnp.float32),
    )(ef, w1, b1, w2, b2)


def _out_proj(p0, p1, wpo, bpo, wout, bout):
    return pl.pallas_call(
        _out_proj_body,
        grid=(N // RN,),
        in_specs=[
            pl.BlockSpec((RN, H), lambda i: (i, 0)),
            pl.BlockSpec((RN, H), lambda i: (i, 0)),
            pl.BlockSpec((H, D), lambda i: (0, 0)),
            pl.BlockSpec((1, D), lambda i: (0, 0)),
            pl.BlockSpec((D, D), lambda i: (0, 0)),
            pl.BlockSpec((1, D), lambda i: (0, 0)),
        ],
        out_specs=pl.BlockSpec((RN, D), lambda i: (i, 0)),
        out_shape=jax.ShapeDtypeStruct((N, D), jnp.float32),
    )(p0, p1, wpo, bpo, wout, bout)


def _sc_gather_mul_scatter(hv, he, src, dst, zinit):
    mesh = plsc.VectorSubcoreMesh(core_axis_name="c", subcore_axis_name="s")

    @functools.partial(
        pl.kernel,
        mesh=mesh,
        out_type=[
            jax.ShapeDtypeStruct((N, H), jnp.float32),
            jax.ShapeDtypeStruct((N, H), jnp.float32),
        ],
        scratch_types=[
            pltpu.VMEM((CH,), jnp.int32),
            pltpu.VMEM((CH,), jnp.int32),
            pltpu.VMEM((CH, H), jnp.float32),
            pltpu.VMEM((CH, H), jnp.float32),
            pltpu.VMEM_SHARED((N, H), jnp.float32),
            pltpu.SemaphoreType.DMA,
        ],
    )
    def k(hv_hbm, he_hbm, src_hbm, dst_hbm, z_hbm, out0, out1,
          src_v, dst_v, he_v, hvr_v, agg_sh, sem):
        c = lax.axis_index("c")
        s = lax.axis_index("s")
        wid = s * NC + c
        row0 = s * ZR
        @pl.when(s < NS - 1)
        def _():
            pltpu.sync_copy(z_hbm.at[pl.ds(row0, ZR)], agg_sh.at[pl.ds(row0, ZR)])

        @pl.when(s == NS - 1)
        def _():
            pltpu.sync_copy(z_hbm.at[pl.ds((NS - 1) * ZR, ZR_LAST)],
                            agg_sh.at[pl.ds((NS - 1) * ZR, ZR_LAST)])

        plsc.subcore_barrier()

        def chunk(it, carry):
            base = wid * EPW + it * CH
            pltpu.sync_copy(src_hbm.at[pl.ds(base, CH)], src_v)
            pltpu.sync_copy(dst_hbm.at[pl.ds(base, CH)], dst_v)
            pltpu.async_copy(hv_hbm.at[src_v], hvr_v, sem).wait()
            pltpu.sync_copy(he_hbm.at[pl.ds(base, CH)], he_v)

            def row(r, cr):
                for j in range(H // 16):
                    sl = pl.ds(j * 16, 16)
                    hvr_v[r, sl] = hvr_v[r, sl] * he_v[r, sl]
                return cr

            lax.fori_loop(0, CH, row, 0)
            pltpu.sync_copy(hvr_v, agg_sh.at[dst_v], add=True)
            return carry

        lax.fori_loop(0, NIT, chunk, 0)
        plsc.subcore_barrier()

        @pl.when((c == 0) & (s < NS - 1))
        def _():
            pltpu.sync_copy(agg_sh.at[pl.ds(row0, ZR)], out0.at[pl.ds(row0, ZR)])

        @pl.when((c == 0) & (s == NS - 1))
        def _():
            pltpu.sync_copy(agg_sh.at[pl.ds((NS - 1) * ZR, ZR_LAST)],
                            out0.at[pl.ds((NS - 1) * ZR, ZR_LAST)])

        @pl.when((c == 1) & (s < NS - 1))
        def _():
            pltpu.sync_copy(agg_sh.at[pl.ds(row0, ZR)], out1.at[pl.ds(row0, ZR)])

        @pl.when((c == 1) & (s == NS - 1))
        def _():
            pltpu.sync_copy(agg_sh.at[pl.ds((NS - 1) * ZR, ZR_LAST)],
                            out1.at[pl.ds((NS - 1) * ZR, ZR_LAST)])

    return k(hv, he, src, dst, zinit)


def kernel(node_feats, edge_feats, edge_index, Wpe1, bpe1, Wpe2, bpe2,
           Wpn, bpn, Wpo, bpo, Wout, bout):
    src = edge_index[0]
    dst = edge_index[1]
    hv = _node_proj(node_feats, Wpn, bpn.reshape(1, H))
    he = _edge_mlp(edge_feats, Wpe1, bpe1.reshape(1, H), Wpe2, bpe2.reshape(1, H))
    zinit = jnp.zeros((N, H), jnp.float32)
    p0, p1 = _sc_gather_mul_scatter(hv, he, src, dst, zinit)
    return _out_proj(p0, p1, Wpo, bpo.reshape(1, H), Wout, bout.reshape(1, D))


# speedup vs baseline: 3.5098x; 1.5881x over previous
import functools

import jax
import jax.numpy as jnp
from jax import lax
from jax.experimental import pallas as pl
from jax.experimental.pallas import tpu as pltpu
from jax.experimental.pallas import tpu_sc as plsc

N = 10000
E = 320000
D = 128
DE = 16
H = 128

NC = 2
NS = 16
NW = NC * NS
EPW = E // NW
CH = 40
NIT = EPW // CH
ZR = 624
ZR_LAST = N - (NS - 1) * ZR

_LOG2 = 0.6931471805599453


def _ssp(x):
    return jnp.maximum(x, 0.0) + jnp.log1p(jnp.exp(-jnp.abs(x))) - _LOG2


def _node_proj_body(nf_ref, w_ref, b_ref, out_ref):
    out_ref[...] = (
        jnp.dot(nf_ref[...], w_ref[...], preferred_element_type=jnp.float32)
        + b_ref[...]
    )


def _edge_mlp_body(ef_ref, w1_ref, b1_ref, w2_ref, b2_ref, out_ref):
    t = jnp.dot(ef_ref[...], w1_ref[...], preferred_element_type=jnp.float32)
    t = _ssp(t + b1_ref[...])
    t = jnp.dot(t, w2_ref[...], preferred_element_type=jnp.float32)
    out_ref[...] = _ssp(t + b2_ref[...])


def _out_proj_body(p0_ref, p1_ref, wpo_ref, bpo_ref, wout_ref, bout_ref, out_ref):
    agg = p0_ref[...] + p1_ref[...]
    h = _ssp(
        jnp.dot(agg, wpo_ref[...], preferred_element_type=jnp.float32)
        + bpo_ref[...]
    )
    out_ref[...] = (
        jnp.dot(h, wout_ref[...], preferred_element_type=jnp.float32)
        + bout_ref[...]
    )


RN = 2000
RE = 2000


def _node_proj(nf, w, b):
    return pl.pallas_call(
        _node_proj_body,
        grid=(N // RN,),
        in_specs=[
            pl.BlockSpec((RN, D), lambda i: (i, 0)),
            pl.BlockSpec((D, H), lambda i: (0, 0)),
            pl.BlockSpec((1, H), lambda i: (0, 0)),
        ],
        out_specs=pl.BlockSpec((RN, H), lambda i: (i, 0)),
        out_shape=jax.ShapeDtypeStruct((N, H), jnp.float32),
    )(nf, w, b)


def _edge_mlp(ef, w1, b1, w2, b2):
    return pl.pallas_call(
        _edge_mlp_body,
        grid=(E // RE,),
        in_specs=[
            pl.BlockSpec((RE, DE), lambda i: (i, 0)),
            pl.BlockSpec((DE, H), lambda i: (0, 0)),
            pl.BlockSpec((1, H), lambda i: (0, 0)),
            pl.BlockSpec((H, H), lambda i: (0, 0)),
            pl.BlockSpec((1, H), lambda i: (0, 0)),
        ],
        out_specs=pl.BlockSpec((RE, H), lambda i: (i, 0)),
        out_shape=jax.ShapeDtypeStruct((E, H), jnp.float32),
    )(ef, w1, b1, w2, b2)


def _out_proj(p0, p1, wpo, bpo, wout, bout):
    return pl.pallas_call(
        _out_proj_body,
        grid=(N // RN,),
        in_specs=[
            pl.BlockSpec((RN, H), lambda i: (i, 0)),
            pl.BlockSpec((RN, H), lambda i: (i, 0)),
            pl.BlockSpec((H, D), lambda i: (0, 0)),
            pl.BlockSpec((1, D), lambda i: (0, 0)),
            pl.BlockSpec((D, D), lambda i: (0, 0)),
            pl.BlockSpec((1, D), lambda i: (0, 0)),
        ],
        out_specs=pl.BlockSpec((RN, D), lambda i: (i, 0)),
        out_shape=jax.ShapeDtypeStruct((N, D), jnp.float32),
    )(p0, p1, wpo, bpo, wout, bout)


def _sc_gather_mul_scatter(hv, he, src3, dst3, zinit):
    mesh = plsc.VectorSubcoreMesh(core_axis_name="c", subcore_axis_name="s")

    @functools.partial(
        pl.kernel,
        mesh=mesh,
        out_type=[
            jax.ShapeDtypeStruct((N, H), jnp.float32),
            jax.ShapeDtypeStruct((N, H), jnp.float32),
        ],
        scratch_types=[
            pltpu.VMEM((EPW,), jnp.int32),
            pltpu.VMEM((CH,), jnp.int32),
            pltpu.VMEM((CH,), jnp.int32),
            pltpu.VMEM((CH, H), jnp.float32),
            pltpu.VMEM((CH, H), jnp.float32),
            pltpu.VMEM((CH, H), jnp.float32),
            pltpu.VMEM((CH, H), jnp.float32),
            pltpu.VMEM((CH, H), jnp.float32),
            pltpu.VMEM_SHARED((N, H), jnp.float32),
            pltpu.SemaphoreType.DMA,
            pltpu.SemaphoreType.DMA,
            pltpu.SemaphoreType.DMA,
            pltpu.SemaphoreType.DMA,
            pltpu.SemaphoreType.DMA,
        ],
    )
    def k(hv_hbm, he_hbm, src_hbm, dst_hbm, z_hbm, out0, out1,
          src_i, d0, d1, he0, he1, hvr0, hvr1, pr, agg_sh,
          g0, g1, h0, h1, s0):
        c = lax.axis_index("c")
        s = lax.axis_index("s")
        wid = s * NC + c
        row0 = s * ZR
        ebase = wid * EPW

        def drain(sem, buf):
            pltpu.make_async_copy(he_hbm.at[pl.ds(0, CH)], buf, sem).wait()

        @pl.when(s < NS - 1)
        def _():
            pltpu.sync_copy(z_hbm.at[pl.ds(row0, ZR)], agg_sh.at[pl.ds(row0, ZR)])

        @pl.when(s == NS - 1)
        def _():
            pltpu.sync_copy(z_hbm.at[pl.ds((NS - 1) * ZR, ZR_LAST)],
                            agg_sh.at[pl.ds((NS - 1) * ZR, ZR_LAST)])

        pltpu.sync_copy(src_hbm.at[pl.ds(ebase, EPW)], src_i)
        plsc.subcore_barrier()

        pltpu.async_copy(hv_hbm.at[src_i.at[pl.ds(0, CH)]], hvr0, g0)
        pltpu.async_copy(he_hbm.at[pl.ds(ebase, CH)], he0, h0)
        pltpu.async_copy(dst_hbm.at[pl.ds(ebase, CH)], d0, h0)

        def process(i, first, hvr, he_b, d_b, gsem, hsem, n_hvr, n_he, n_d, n_g, n_h):
            @pl.when(i + 1 < NIT)
            def _():
                pltpu.async_copy(
                    hv_hbm.at[src_i.at[pl.ds((i + 1) * CH, CH)]], n_hvr, n_g)
                pltpu.async_copy(he_hbm.at[pl.ds(ebase + (i + 1) * CH, CH)],
                                 n_he, n_h)
                pltpu.async_copy(dst_hbm.at[pl.ds(ebase + (i + 1) * CH, CH)],
                                 n_d, n_h)

            drain(gsem, hvr)
            drain(hsem, he_b)
            pltpu.make_async_copy(dst_hbm.at[pl.ds(0, CH)], d_b, hsem).wait()

            @pl.when(jnp.logical_not(first))
            def _():
                drain(s0, pr)

            def row(r, cr):
                for j in range(H // 16):
                    sl = pl.ds(j * 16, 16)
                    pr[r, sl] = hvr[r, sl] * he_b[r, sl]
                return cr

            lax.fori_loop(0, CH, row, 0)
            pltpu.async_copy(pr, agg_sh.at[d_b], s0, add=True)

        def outer(io, carry):
            i0 = io * 2
            process(i0, io == 0, hvr0, he0, d0, g0, h0, hvr1, he1, d1, g1, h1)
            process(i0 + 1, jnp.bool_(False), hvr1, he1, d1, g1, h1,
                    hvr0, he0, d0, g0, h0)
            return carry

        lax.fori_loop(0, NIT // 2, outer, 0)
        drain(s0, pr)
        plsc.subcore_barrier()

        @pl.when((c == 0) & (s < NS - 1))
        def _():
            pltpu.sync_copy(agg_sh.at[pl.ds(row0, ZR)], out0.at[pl.ds(row0, ZR)])

        @pl.when((c == 0) & (s == NS - 1))
        def _():
            pltpu.sync_copy(agg_sh.at[pl.ds((NS - 1) * ZR, ZR_LAST)],
                            out0.at[pl.ds((NS - 1) * ZR, ZR_LAST)])

        @pl.when((c == 1) & (s < NS - 1))
        def _():
            pltpu.sync_copy(agg_sh.at[pl.ds(row0, ZR)], out1.at[pl.ds(row0, ZR)])

        @pl.when((c == 1) & (s == NS - 1))
        def _():
            pltpu.sync_copy(agg_sh.at[pl.ds((NS - 1) * ZR, ZR_LAST)],
                            out1.at[pl.ds((NS - 1) * ZR, ZR_LAST)])

    return k(hv, he, src3, dst3, zinit)


def kernel(node_feats, edge_feats, edge_index, Wpe1, bpe1, Wpe2, bpe2,
           Wpn, bpn, Wpo, bpo, Wout, bout):
    src3 = edge_index[0]
    dst3 = edge_index[1]
    hv = _node_proj(node_feats, Wpn, bpn.reshape(1, H))
    he = _edge_mlp(edge_feats, Wpe1, bpe1.reshape(1, H), Wpe2, bpe2.reshape(1, H))
    zinit = jnp.zeros((N, H), jnp.float32)
    p0, p1 = _sc_gather_mul_scatter(hv, he, src3, dst3, zinit)
    return _out_proj(p0, p1, Wpo, bpo.reshape(1, H), Wout, bout.reshape(1, D))


# speedup vs baseline: 4.3567x; 1.2413x over previous
import functools

import jax
import jax.numpy as jnp
from jax import lax
from jax.experimental import pallas as pl
from jax.experimental.pallas import tpu as pltpu
from jax.experimental.pallas import tpu_sc as plsc

N = 10000
E = 320000
D = 128
DE = 16
H = 128

NC = 2
NS = 16
NW = NC * NS
EPW = E // NW
CH = 40
NIT = EPW // CH
ZR = 624
ZR_LAST = N - (NS - 1) * ZR

_LOG2 = 0.6931471805599453


def _ssp(x):
    return jnp.maximum(x, 0.0) + jnp.log1p(jnp.exp(-jnp.abs(x))) - _LOG2


def _node_proj_body(nf_ref, w_ref, b_ref, out_ref):
    out_ref[...] = (
        jnp.dot(nf_ref[...], w_ref[...], preferred_element_type=jnp.float32)
        + b_ref[...]
    )


def _edge_mlp_body(eft_ref, w1_ref, b1_ref, w2_ref, b2_ref, out_ref):
    t = lax.dot_general(eft_ref[...], w1_ref[...], (((0,), (0,)), ((), ())),
                        preferred_element_type=jnp.float32)
    t = _ssp(t + b1_ref[...])
    t = jnp.dot(t, w2_ref[...], preferred_element_type=jnp.float32)
    out_ref[...] = _ssp(t + b2_ref[...])


def _out_proj_body(p0_ref, p1_ref, wpo_ref, bpo_ref, wout_ref, bout_ref, out_ref):
    agg = p0_ref[...] + p1_ref[...]
    h = _ssp(
        jnp.dot(agg, wpo_ref[...], preferred_element_type=jnp.float32)
        + bpo_ref[...]
    )
    out_ref[...] = (
        jnp.dot(h, wout_ref[...], preferred_element_type=jnp.float32)
        + bout_ref[...]
    )


RN = 2000
RE = 2560


def _node_proj(nf, w, b):
    return pl.pallas_call(
        _node_proj_body,
        grid=(N // RN,),
        in_specs=[
            pl.BlockSpec((RN, D), lambda i: (i, 0)),
            pl.BlockSpec((D, H), lambda i: (0, 0)),
            pl.BlockSpec((1, H), lambda i: (0, 0)),
        ],
        out_specs=pl.BlockSpec((RN, H), lambda i: (i, 0)),
        out_shape=jax.ShapeDtypeStruct((N, H), jnp.float32),
    )(nf, w, b)


def _edge_mlp(eft, w1, b1, w2, b2):
    return pl.pallas_call(
        _edge_mlp_body,
        grid=(E // RE,),
        in_specs=[
            pl.BlockSpec((DE, RE), lambda i: (0, i)),
            pl.BlockSpec((DE, H), lambda i: (0, 0)),
            pl.BlockSpec((1, H), lambda i: (0, 0)),
            pl.BlockSpec((H, H), lambda i: (0, 0)),
            pl.BlockSpec((1, H), lambda i: (0, 0)),
        ],
        out_specs=pl.BlockSpec((RE, H), lambda i: (i, 0)),
        out_shape=jax.ShapeDtypeStruct((E, H), jnp.float32),
    )(eft, w1, b1, w2, b2)


def _out_proj(p0, p1, wpo, bpo, wout, bout):
    return pl.pallas_call(
        _out_proj_body,
        grid=(N // RN,),
        in_specs=[
            pl.BlockSpec((RN, H), lambda i: (i, 0)),
            pl.BlockSpec((RN, H), lambda i: (i, 0)),
            pl.BlockSpec((H, D), lambda i: (0, 0)),
            pl.BlockSpec((1, D), lambda i: (0, 0)),
            pl.BlockSpec((D, D), lambda i: (0, 0)),
            pl.BlockSpec((1, D), lambda i: (0, 0)),
        ],
        out_specs=pl.BlockSpec((RN, D), lambda i: (i, 0)),
        out_shape=jax.ShapeDtypeStruct((N, D), jnp.float32),
    )(p0, p1, wpo, bpo, wout, bout)


def _sc_gather_mul_scatter(hv, he, src3, dst3, zinit):
    mesh = plsc.VectorSubcoreMesh(core_axis_name="c", subcore_axis_name="s")

    @functools.partial(
        pl.kernel,
        mesh=mesh,
        out_type=[
            jax.ShapeDtypeStruct((N, H), jnp.float32),
            jax.ShapeDtypeStruct((N, H), jnp.float32),
        ],
        scratch_types=[
            pltpu.VMEM((EPW,), jnp.int32),
            pltpu.VMEM((CH,), jnp.int32),
            pltpu.VMEM((CH,), jnp.int32),
            pltpu.VMEM((CH, H), jnp.float32),
            pltpu.VMEM((CH, H), jnp.float32),
            pltpu.VMEM((CH, H), jnp.float32),
            pltpu.VMEM((CH, H), jnp.float32),
            pltpu.VMEM((CH, H), jnp.float32),
            pltpu.VMEM_SHARED((N, H), jnp.float32),
            pltpu.SemaphoreType.DMA,
            pltpu.SemaphoreType.DMA,
            pltpu.SemaphoreType.DMA,
            pltpu.SemaphoreType.DMA,
            pltpu.SemaphoreType.DMA,
        ],
    )
    def k(hv_hbm, he_hbm, src_hbm, dst_hbm, z_hbm, out0, out1,
          src_i, d0, d1, he0, he1, hvr0, hvr1, pr, agg_sh,
          g0, g1, h0, h1, s0):
        c = lax.axis_index("c")
        s = lax.axis_index("s")
        wid = s * NC + c
        row0 = s * ZR
        ebase = wid * EPW

        def drain(sem, buf):
            pltpu.make_async_copy(he_hbm.at[pl.ds(0, CH)], buf, sem).wait()

        @pl.when(s < NS - 1)
        def _():
            pltpu.sync_copy(z_hbm.at[pl.ds(row0, ZR)], agg_sh.at[pl.ds(row0, ZR)])

        @pl.when(s == NS - 1)
        def _():
            pltpu.sync_copy(z_hbm.at[pl.ds((NS - 1) * ZR, ZR_LAST)],
                            agg_sh.at[pl.ds((NS - 1) * ZR, ZR_LAST)])

        pltpu.sync_copy(src_hbm.at[pl.ds(ebase, EPW)], src_i)
        plsc.subcore_barrier()

        pltpu.async_copy(hv_hbm.at[src_i.at[pl.ds(0, CH)]], hvr0, g0)
        pltpu.async_copy(he_hbm.at[pl.ds(ebase, CH)], he0, h0)
        pltpu.async_copy(dst_hbm.at[pl.ds(ebase, CH)], d0, h0)

        def process(i, first, hvr, he_b, d_b, gsem, hsem, n_hvr, n_he, n_d, n_g, n_h):
            @pl.when(i + 1 < NIT)
            def _():
                pltpu.async_copy(
                    hv_hbm.at[src_i.at[pl.ds((i + 1) * CH, CH)]], n_hvr, n_g)
                pltpu.async_copy(he_hbm.at[pl.ds(ebase + (i + 1) * CH, CH)],
                                 n_he, n_h)
                pltpu.async_copy(dst_hbm.at[pl.ds(ebase + (i + 1) * CH, CH)],
                                 n_d, n_h)

            drain(gsem, hvr)
            drain(hsem, he_b)
            pltpu.make_async_copy(dst_hbm.at[pl.ds(0, CH)], d_b, hsem).wait()

            @pl.when(jnp.logical_not(first))
            def _():
                drain(s0, pr)

            def row(r, cr):
                for j in range(H // 16):
                    sl = pl.ds(j * 16, 16)
                    pr[r, sl] = hvr[r, sl] * he_b[r, sl]
                return cr

            lax.fori_loop(0, CH, row, 0)
            pltpu.async_copy(pr, agg_sh.at[d_b], s0, add=True)

        def outer(io, carry):
            i0 = io * 2
            process(i0, io == 0, hvr0, he0, d0, g0, h0, hvr1, he1, d1, g1, h1)
            process(i0 + 1, jnp.bool_(False), hvr1, he1, d1, g1, h1,
                    hvr0, he0, d0, g0, h0)
            return carry

        lax.fori_loop(0, NIT // 2, outer, 0)
        drain(s0, pr)
        plsc.subcore_barrier()

        @pl.when((c == 0) & (s < NS - 1))
        def _():
            pltpu.sync_copy(agg_sh.at[pl.ds(row0, ZR)], out0.at[pl.ds(row0, ZR)])

        @pl.when((c == 0) & (s == NS - 1))
        def _():
            pltpu.sync_copy(agg_sh.at[pl.ds((NS - 1) * ZR, ZR_LAST)],
                            out0.at[pl.ds((NS - 1) * ZR, ZR_LAST)])

        @pl.when((c == 1) & (s < NS - 1))
        def _():
            pltpu.sync_copy(agg_sh.at[pl.ds(row0, ZR)], out1.at[pl.ds(row0, ZR)])

        @pl.when((c == 1) & (s == NS - 1))
        def _():
            pltpu.sync_copy(agg_sh.at[pl.ds((NS - 1) * ZR, ZR_LAST)],
                            out1.at[pl.ds((NS - 1) * ZR, ZR_LAST)])

    return k(hv, he, src3, dst3, zinit)


def kernel(node_feats, edge_feats, edge_index, Wpe1, bpe1, Wpe2, bpe2,
           Wpn, bpn, Wpo, bpo, Wout, bout):
    src3 = edge_index[0]
    dst3 = edge_index[1]
    hv = _node_proj(node_feats, Wpn, bpn.reshape(1, H))
    he = _edge_mlp(edge_feats.T, Wpe1, bpe1.reshape(1, H), Wpe2, bpe2.reshape(1, H))
    zinit = jnp.zeros((N, H), jnp.float32)
    p0, p1 = _sc_gather_mul_scatter(hv, he, src3, dst3, zinit)
    return _out_proj(p0, p1, Wpo, bpo.reshape(1, H), Wout, bout.reshape(1, D))
